# baseline (speedup 1.0000x reference)
import functools

import jax
import jax.numpy as jnp
from jax import lax
from jax.experimental import pallas as pl
from jax.experimental.pallas import tpu as pltpu
from jax.experimental.pallas import tpu_sc as plsc

_D = 64
_W = 2 * _D
_L = 16
_CHUNK = 128
_NCHUNK = 4
_ROWS_PER_TILE = _CHUNK * _NCHUNK
_GLOBAL_MEAN = 3.5
_TB = 1024

_info = plsc.get_sparse_core_info()
_NC = _info.num_cores
_NS = _info.num_subcores
_NW = _NC * _NS


def _cat_body(ut_ref, it_ref, o_ref):
    eye = jnp.eye(_D, dtype=jnp.float32)
    dn = (((0,), (0,)), ((), ()))
    o_ref[:, 0:_D] = lax.dot_general(
        ut_ref[...], eye, dn, preferred_element_type=jnp.float32)
    o_ref[:, _D:_W] = lax.dot_general(
        it_ref[...], eye, dn, preferred_element_type=jnp.float32)


def _make_cat(n_rows):
    grid = pl.cdiv(n_rows, _TB)
    return pl.pallas_call(
        _cat_body,
        grid=(grid,),
        in_specs=[
            pl.BlockSpec((_D, _TB), lambda j: (0, j)),
            pl.BlockSpec((_D, _TB), lambda j: (0, j)),
        ],
        out_specs=pl.BlockSpec((_TB, _W), lambda j: (j, 0)),
        out_shape=jax.ShapeDtypeStruct((n_rows, _W), jnp.float32),
    )


def _mf_body(uids_r, iids_r, cat, u_biases, i_biases,
             out_hbm, uidx_v, iidx_v, bidx_u, bidx_i,
             u_rows, i_rows, ub_rows, ib_rows, out_v, sem0, sem1):
    wid = lax.axis_index("s") * _NC + lax.axis_index("c")

    pltpu.sync_copy(uids_r.at[wid], uidx_v)
    pltpu.sync_copy(iids_r.at[wid], iidx_v)

    lane = lax.broadcasted_iota(jnp.int32, (_L,), 0)

    def compute_bidx(c):
        cc = jnp.full((_L,), c, jnp.int32)

        def bidx_body(g, carry):
            rows = g * _L + lane
            uq = plsc.load_gather(uidx_v, [cc, rows])
            iq = plsc.load_gather(iidx_v, [cc, rows])
            bidx_u[c, pl.ds(g * _L, _L)] = uq >> 4
            bidx_i[c, pl.ds(g * _L, _L)] = iq >> 4
            return carry

        lax.fori_loop(0, _CHUNK // _L, bidx_body, 0)

    def issue(c, buf):
        sem = sem0 if buf == 0 else sem1
        cp_u = pltpu.async_copy(cat.at[uidx_v.at[c]], u_rows.at[buf], sem)
        cp_i = pltpu.async_copy(cat.at[iidx_v.at[c]], i_rows.at[buf], sem)
        cp_ub = pltpu.async_copy(u_biases.at[bidx_u.at[c]], ub_rows.at[buf], sem)
        cp_ib = pltpu.async_copy(i_biases.at[bidx_i.at[c]], ib_rows.at[buf], sem)
        return (cp_u, cp_i, cp_ub, cp_ib)

    def compute(c, buf):
        cc = jnp.full((_L,), c, jnp.int32)
        ur = u_rows.at[buf]
        ir = i_rows.at[buf]
        ubr = ub_rows.at[buf]
        ibr = ib_rows.at[buf]

        def group_body(g, carry):
            rows = g * _L + lane
            a0 = jnp.zeros((_L,), jnp.float32)
            a1 = jnp.zeros((_L,), jnp.float32)
            a2 = jnp.zeros((_L,), jnp.float32)
            a3 = jnp.zeros((_L,), jnp.float32)
            for d in range(0, _D, 4):
                c0 = (lane + d) & 63
                c1 = (lane + d + 1) & 63
                c2 = (lane + d + 2) & 63
                c3 = (lane + d + 3) & 63
                a0 = a0 + (plsc.load_gather(ur, [rows, c0]) *
                           plsc.load_gather(ir, [rows, c0 + _D]))
                a1 = a1 + (plsc.load_gather(ur, [rows, c1]) *
                           plsc.load_gather(ir, [rows, c1 + _D]))
                a2 = a2 + (plsc.load_gather(ur, [rows, c2]) *
                           plsc.load_gather(ir, [rows, c2 + _D]))
                a3 = a3 + (plsc.load_gather(ur, [rows, c3]) *
                           plsc.load_gather(ir, [rows, c3 + _D]))
            uq = plsc.load_gather(uidx_v, [cc, rows])
            iq = plsc.load_gather(iidx_v, [cc, rows])
            acc = ((a0 + a1) + (a2 + a3)
                   + plsc.load_gather(ubr, [rows, uq & 15])
                   + plsc.load_gather(ibr, [rows, iq & 15]) + _GLOBAL_MEAN)
            out_v[pl.ds(c * _CHUNK + g * _L, _L)] = acc
            return carry

        lax.fori_loop(0, _CHUNK // _L, group_body, 0)

    compute_bidx(0)
    cps = issue(0, 0)
    for c in range(_NCHUNK):
        nxt = c + 1
        if nxt < _NCHUNK:
            compute_bidx(nxt)
            nxt_cps = issue(nxt, nxt % 2)
        for cp in cps:
            cp.wait()
        compute(c, c % 2)
        if nxt < _NCHUNK:
            cps = nxt_cps

    pltpu.sync_copy(out_v, out_hbm.at[wid])


_mf_call = functools.partial(
    pl.kernel,
    out_type=jax.ShapeDtypeStruct((_NW, _ROWS_PER_TILE), jnp.float32),
    mesh=plsc.VectorSubcoreMesh(core_axis_name="c", subcore_axis_name="s"),
    compiler_params=pltpu.CompilerParams(
        needs_layout_passes=False, use_tc_tiling_on_sc=False),
    scratch_types=[
        pltpu.VMEM((_NCHUNK, _CHUNK), jnp.int32),
        pltpu.VMEM((_NCHUNK, _CHUNK), jnp.int32),
        pltpu.VMEM((_NCHUNK, _CHUNK), jnp.int32),
        pltpu.VMEM((_NCHUNK, _CHUNK), jnp.int32),
        pltpu.VMEM((2, _CHUNK, _W), jnp.float32),
        pltpu.VMEM((2, _CHUNK, _W), jnp.float32),
        pltpu.VMEM((2, _CHUNK, _L), jnp.float32),
        pltpu.VMEM((2, _CHUNK, _L), jnp.float32),
        pltpu.VMEM((_ROWS_PER_TILE,), jnp.float32),
        pltpu.SemaphoreType.DMA,
        pltpu.SemaphoreType.DMA,
    ],
)(_mf_body)


@jax.jit
def kernel(uids, iids, u_factors, i_factors, u_biases, i_biases):
    uids_r = uids.astype(jnp.int32).reshape(_NW, _NCHUNK, _CHUNK)
    iids_r = iids.astype(jnp.int32).reshape(_NW, _NCHUNK, _CHUNK)
    cat = _make_cat(u_factors.shape[0])(u_factors.T, i_factors.T)
    ub_r = u_biases.reshape(-1, _L)
    ib_r = i_biases.reshape(-1, _L)
    out = _mf_call(uids_r, iids_r, cat, ub_r, ib_r)
    return out.reshape(-1)


# speedup vs baseline: 1.2458x; 1.2458x over previous
import functools

import jax
import jax.numpy as jnp
from jax import lax
from jax.experimental import pallas as pl
from jax.experimental.pallas import tpu as pltpu
from jax.experimental.pallas import tpu_sc as plsc

_D = 64
_W = 2 * _D
_L = 16
_CHUNK = 128
_NCHUNK = 4
_ROWS_PER_TILE = _CHUNK * _NCHUNK
_GLOBAL_MEAN = 3.5
_TB = 2048

_info = plsc.get_sparse_core_info()
_NC = _info.num_cores
_NS = _info.num_subcores
_NW = _NC * _NS


def _cat_body(ut_ref, it_ref, o_ref):
    o_ref[:, 0:_D] = ut_ref[...].T
    o_ref[:, _D:_W] = it_ref[...].T


def _make_cat(n_rows):
    grid = pl.cdiv(n_rows, _TB)
    return pl.pallas_call(
        _cat_body,
        grid=(grid,),
        in_specs=[
            pl.BlockSpec((_D, _TB), lambda j: (0, j)),
            pl.BlockSpec((_D, _TB), lambda j: (0, j)),
        ],
        out_specs=pl.BlockSpec((_TB, _W), lambda j: (j, 0)),
        out_shape=jax.ShapeDtypeStruct((n_rows, _W), jnp.float32),
    )


def _mf_body(uids_r, iids_r, cat, u_biases, i_biases,
             out_hbm, uidx_v, iidx_v, bidx_u, bidx_i,
             u_rows, i_rows, ub_rows, ib_rows, out_v, sem0, sem1):
    wid = lax.axis_index("s") * _NC + lax.axis_index("c")

    pltpu.sync_copy(uids_r.at[wid], uidx_v)
    pltpu.sync_copy(iids_r.at[wid], iidx_v)

    lane = lax.broadcasted_iota(jnp.int32, (_L,), 0)

    def compute_bidx(c):
        cc = jnp.full((_L,), c, jnp.int32)

        def bidx_body(g, carry):
            rows = g * _L + lane
            uq = plsc.load_gather(uidx_v, [cc, rows])
            iq = plsc.load_gather(iidx_v, [cc, rows])
            bidx_u[c, pl.ds(g * _L, _L)] = uq >> 4
            bidx_i[c, pl.ds(g * _L, _L)] = iq >> 4
            return carry

        lax.fori_loop(0, _CHUNK // _L, bidx_body, 0)

    def issue(c, buf):
        sem = sem0 if buf == 0 else sem1
        cp_u = pltpu.async_copy(cat.at[uidx_v.at[c]], u_rows.at[buf], sem)
        cp_i = pltpu.async_copy(cat.at[iidx_v.at[c]], i_rows.at[buf], sem)
        cp_ub = pltpu.async_copy(u_biases.at[bidx_u.at[c]], ub_rows.at[buf], sem)
        cp_ib = pltpu.async_copy(i_biases.at[bidx_i.at[c]], ib_rows.at[buf], sem)
        return (cp_u, cp_i, cp_ub, cp_ib)

    def compute(c, buf):
        cc = jnp.full((_L,), c, jnp.int32)
        ur = u_rows.at[buf]
        ir = i_rows.at[buf]
        ubr = ub_rows.at[buf]
        ibr = ib_rows.at[buf]

        def group_body(g, carry):
            rows = g * _L + lane
            a0 = jnp.zeros((_L,), jnp.float32)
            a1 = jnp.zeros((_L,), jnp.float32)
            a2 = jnp.zeros((_L,), jnp.float32)
            a3 = jnp.zeros((_L,), jnp.float32)
            for d in range(0, _D, 4):
                c0 = (lane + d) & 63
                c1 = (lane + d + 1) & 63
                c2 = (lane + d + 2) & 63
                c3 = (lane + d + 3) & 63
                a0 = a0 + (plsc.load_gather(ur, [rows, c0]) *
                           plsc.load_gather(ir, [rows, c0 + _D]))
                a1 = a1 + (plsc.load_gather(ur, [rows, c1]) *
                           plsc.load_gather(ir, [rows, c1 + _D]))
                a2 = a2 + (plsc.load_gather(ur, [rows, c2]) *
                           plsc.load_gather(ir, [rows, c2 + _D]))
                a3 = a3 + (plsc.load_gather(ur, [rows, c3]) *
                           plsc.load_gather(ir, [rows, c3 + _D]))
            uq = plsc.load_gather(uidx_v, [cc, rows])
            iq = plsc.load_gather(iidx_v, [cc, rows])
            acc = ((a0 + a1) + (a2 + a3)
                   + plsc.load_gather(ubr, [rows, uq & 15])
                   + plsc.load_gather(ibr, [rows, iq & 15]) + _GLOBAL_MEAN)
            out_v[pl.ds(c * _CHUNK + g * _L, _L)] = acc
            return carry

        lax.fori_loop(0, _CHUNK // _L, group_body, 0)

    compute_bidx(0)
    cps = issue(0, 0)
    for c in range(_NCHUNK):
        nxt = c + 1
        if nxt < _NCHUNK:
            compute_bidx(nxt)
            nxt_cps = issue(nxt, nxt % 2)
        for cp in cps:
            cp.wait()
        compute(c, c % 2)
        if nxt < _NCHUNK:
            cps = nxt_cps

    pltpu.sync_copy(out_v, out_hbm.at[wid])


_mf_call = functools.partial(
    pl.kernel,
    out_type=jax.ShapeDtypeStruct((_NW, _ROWS_PER_TILE), jnp.float32),
    mesh=plsc.VectorSubcoreMesh(core_axis_name="c", subcore_axis_name="s"),
    compiler_params=pltpu.CompilerParams(
        needs_layout_passes=False, use_tc_tiling_on_sc=False),
    scratch_types=[
        pltpu.VMEM((_NCHUNK, _CHUNK), jnp.int32),
        pltpu.VMEM((_NCHUNK, _CHUNK), jnp.int32),
        pltpu.VMEM((_NCHUNK, _CHUNK), jnp.int32),
        pltpu.VMEM((_NCHUNK, _CHUNK), jnp.int32),
        pltpu.VMEM((2, _CHUNK, _W), jnp.float32),
        pltpu.VMEM((2, _CHUNK, _W), jnp.float32),
        pltpu.VMEM((2, _CHUNK, _L), jnp.float32),
        pltpu.VMEM((2, _CHUNK, _L), jnp.float32),
        pltpu.VMEM((_ROWS_PER_TILE,), jnp.float32),
        pltpu.SemaphoreType.DMA,
        pltpu.SemaphoreType.DMA,
    ],
)(_mf_body)


@jax.jit
def kernel(uids, iids, u_factors, i_factors, u_biases, i_biases):
    uids_r = uids.astype(jnp.int32).reshape(_NW, _NCHUNK, _CHUNK)
    iids_r = iids.astype(jnp.int32).reshape(_NW, _NCHUNK, _CHUNK)
    cat = _make_cat(u_factors.shape[0])(u_factors.T, i_factors.T)
    ub_r = u_biases.reshape(-1, _L)
    ib_r = i_biases.reshape(-1, _L)
    out = _mf_call(uids_r, iids_r, cat, ub_r, ib_r)
    return out.reshape(-1)


# speedup vs baseline: 1.4112x; 1.1327x over previous
import functools

import jax
import jax.numpy as jnp
from jax import lax
from jax.experimental import pallas as pl
from jax.experimental.pallas import tpu as pltpu
from jax.experimental.pallas import tpu_sc as plsc

_D = 64
_W = 2 * _D
_L = 16
_CHUNK = 128
_NCHUNK = 4
_ROWS_PER_TILE = _CHUNK * _NCHUNK
_GLOBAL_MEAN = 3.5
_TB = 4096

_info = plsc.get_sparse_core_info()
_NC = _info.num_cores
_NS = _info.num_subcores
_NW = _NC * _NS


def _cat_body(ut_ref, it_ref, o_ref):
    o_ref[:, 0:_D] = ut_ref[...].T
    o_ref[:, _D:_W] = it_ref[...].T


def _make_cat(n_rows):
    grid = pl.cdiv(n_rows, _TB)
    return pl.pallas_call(
        _cat_body,
        grid=(grid,),
        in_specs=[
            pl.BlockSpec((_D, _TB), lambda j: (0, j)),
            pl.BlockSpec((_D, _TB), lambda j: (0, j)),
        ],
        out_specs=pl.BlockSpec((_TB, _W), lambda j: (j, 0)),
        out_shape=jax.ShapeDtypeStruct((n_rows, _W), jnp.float32),
    )


def _mf_body(uids_r, iids_r, cat, u_biases, i_biases,
             out_hbm, uidx_v, iidx_v, bidx_u, bidx_i,
             u_rows, i_rows, ub_rows, ib_rows, out_v, sem0, sem1):
    wid = lax.axis_index("s") * _NC + lax.axis_index("c")

    pltpu.sync_copy(uids_r.at[wid], uidx_v)
    pltpu.sync_copy(iids_r.at[wid], iidx_v)

    lane = lax.broadcasted_iota(jnp.int32, (_L,), 0)

    def compute_bidx(c):
        cc = jnp.full((_L,), c, jnp.int32)

        def bidx_body(g, carry):
            rows = g * _L + lane
            uq = plsc.load_gather(uidx_v, [cc, rows])
            iq = plsc.load_gather(iidx_v, [cc, rows])
            bidx_u[c, pl.ds(g * _L, _L)] = uq >> 4
            bidx_i[c, pl.ds(g * _L, _L)] = iq >> 4
            return carry

        lax.fori_loop(0, _CHUNK // _L, bidx_body, 0)

    def issue(c, buf):
        sem = sem0 if buf == 0 else sem1
        cp_u = pltpu.async_copy(cat.at[uidx_v.at[c]], u_rows.at[buf], sem)
        cp_i = pltpu.async_copy(cat.at[iidx_v.at[c]], i_rows.at[buf], sem)
        cp_ub = pltpu.async_copy(u_biases.at[bidx_u.at[c]], ub_rows.at[buf], sem)
        cp_ib = pltpu.async_copy(i_biases.at[bidx_i.at[c]], ib_rows.at[buf], sem)
        return (cp_u, cp_i, cp_ub, cp_ib)

    def compute(c, buf):
        cc = jnp.full((_L,), c, jnp.int32)
        ur = u_rows.at[buf]
        ir = i_rows.at[buf]
        ubr = ub_rows.at[buf]
        ibr = ib_rows.at[buf]

        def group_body(g, carry):
            rows = g * _L + lane
            a0 = jnp.zeros((_L,), jnp.float32)
            a1 = jnp.zeros((_L,), jnp.float32)
            a2 = jnp.zeros((_L,), jnp.float32)
            a3 = jnp.zeros((_L,), jnp.float32)
            for d in range(0, _D, 4):
                c0 = (lane + d) & 63
                c1 = (lane + d + 1) & 63
                c2 = (lane + d + 2) & 63
                c3 = (lane + d + 3) & 63
                a0 = a0 + (plsc.load_gather(ur, [rows, c0]) *
                           plsc.load_gather(ir, [rows, c0 + _D]))
                a1 = a1 + (plsc.load_gather(ur, [rows, c1]) *
                           plsc.load_gather(ir, [rows, c1 + _D]))
                a2 = a2 + (plsc.load_gather(ur, [rows, c2]) *
                           plsc.load_gather(ir, [rows, c2 + _D]))
                a3 = a3 + (plsc.load_gather(ur, [rows, c3]) *
                           plsc.load_gather(ir, [rows, c3 + _D]))
            uq = plsc.load_gather(uidx_v, [cc, rows])
            iq = plsc.load_gather(iidx_v, [cc, rows])
            acc = ((a0 + a1) + (a2 + a3)
                   + plsc.load_gather(ubr, [rows, uq & 15])
                   + plsc.load_gather(ibr, [rows, iq & 15]) + _GLOBAL_MEAN)
            out_v[pl.ds(c * _CHUNK + g * _L, _L)] = acc
            return carry

        lax.fori_loop(0, _CHUNK // _L, group_body, 0)

    compute_bidx(0)
    cps = issue(0, 0)
    for c in range(_NCHUNK):
        nxt = c + 1
        if nxt < _NCHUNK:
            compute_bidx(nxt)
            nxt_cps = issue(nxt, nxt % 2)
        for cp in cps:
            cp.wait()
        compute(c, c % 2)
        if nxt < _NCHUNK:
            cps = nxt_cps

    pltpu.sync_copy(out_v, out_hbm.at[wid])


_mf_call = functools.partial(
    pl.kernel,
    out_type=jax.ShapeDtypeStruct((_NW, _ROWS_PER_TILE), jnp.float32),
    mesh=plsc.VectorSubcoreMesh(core_axis_name="c", subcore_axis_name="s"),
    compiler_params=pltpu.CompilerParams(
        needs_layout_passes=False, use_tc_tiling_on_sc=False),
    scratch_types=[
        pltpu.VMEM((_NCHUNK, _CHUNK), jnp.int32),
        pltpu.VMEM((_NCHUNK, _CHUNK), jnp.int32),
        pltpu.VMEM((_NCHUNK, _CHUNK), jnp.int32),
        pltpu.VMEM((_NCHUNK, _CHUNK), jnp.int32),
        pltpu.VMEM((2, _CHUNK, _W), jnp.float32),
        pltpu.VMEM((2, _CHUNK, _W), jnp.float32),
        pltpu.VMEM((2, _CHUNK, _L), jnp.float32),
        pltpu.VMEM((2, _CHUNK, _L), jnp.float32),
        pltpu.VMEM((_ROWS_PER_TILE,), jnp.float32),
        pltpu.SemaphoreType.DMA,
        pltpu.SemaphoreType.DMA,
    ],
)(_mf_body)


@jax.jit
def kernel(uids, iids, u_factors, i_factors, u_biases, i_biases):
    uids_r = uids.astype(jnp.int32).reshape(_NW, _NCHUNK, _CHUNK)
    iids_r = iids.astype(jnp.int32).reshape(_NW, _NCHUNK, _CHUNK)
    cat = _make_cat(u_factors.shape[0])(u_factors.T, i_factors.T)
    ub_r = u_biases.reshape(-1, _L)
    ib_r = i_biases.reshape(-1, _L)
    out = _mf_call(uids_r, iids_r, cat, ub_r, ib_r)
    return out.reshape(-1)


# speedup vs baseline: 1.4822x; 1.0503x over previous
import functools

import jax
import jax.numpy as jnp
from jax import lax
from jax.experimental import pallas as pl
from jax.experimental.pallas import tpu as pltpu
from jax.experimental.pallas import tpu_sc as plsc

_D = 64
_W = 2 * _D
_L = 16
_CHUNK = 128
_NCHUNK = 4
_ROWS_PER_TILE = _CHUNK * _NCHUNK
_GLOBAL_MEAN = 3.5
_TB = 8192

_info = plsc.get_sparse_core_info()
_NC = _info.num_cores
_NS = _info.num_subcores
_NW = _NC * _NS


def _cat_body(ut_ref, it_ref, o_ref):
    o_ref[:, 0:_D] = ut_ref[...].T
    o_ref[:, _D:_W] = it_ref[...].T


def _make_cat(n_rows):
    grid = pl.cdiv(n_rows, _TB)
    return pl.pallas_call(
        _cat_body,
        grid=(grid,),
        in_specs=[
            pl.BlockSpec((_D, _TB), lambda j: (0, j)),
            pl.BlockSpec((_D, _TB), lambda j: (0, j)),
        ],
        out_specs=pl.BlockSpec((_TB, _W), lambda j: (j, 0)),
        out_shape=jax.ShapeDtypeStruct((n_rows, _W), jnp.float32),
    )


def _mf_body(uids_r, iids_r, cat, u_biases, i_biases,
             out_hbm, uidx_v, iidx_v, bidx_u, bidx_i,
             u_rows, i_rows, ub_rows, ib_rows, out_v, sem0, sem1):
    wid = lax.axis_index("s") * _NC + lax.axis_index("c")

    pltpu.sync_copy(uids_r.at[wid], uidx_v)
    pltpu.sync_copy(iids_r.at[wid], iidx_v)

    lane = lax.broadcasted_iota(jnp.int32, (_L,), 0)

    def compute_bidx(c):
        cc = jnp.full((_L,), c, jnp.int32)

        def bidx_body(g, carry):
            rows = g * _L + lane
            uq = plsc.load_gather(uidx_v, [cc, rows])
            iq = plsc.load_gather(iidx_v, [cc, rows])
            bidx_u[c, pl.ds(g * _L, _L)] = uq >> 4
            bidx_i[c, pl.ds(g * _L, _L)] = iq >> 4
            return carry

        lax.fori_loop(0, _CHUNK // _L, bidx_body, 0)

    def issue(c, buf):
        sem = sem0 if buf == 0 else sem1
        cp_u = pltpu.async_copy(cat.at[uidx_v.at[c]], u_rows.at[buf], sem)
        cp_i = pltpu.async_copy(cat.at[iidx_v.at[c]], i_rows.at[buf], sem)
        cp_ub = pltpu.async_copy(u_biases.at[bidx_u.at[c]], ub_rows.at[buf], sem)
        cp_ib = pltpu.async_copy(i_biases.at[bidx_i.at[c]], ib_rows.at[buf], sem)
        return (cp_u, cp_i, cp_ub, cp_ib)

    def compute(c, buf):
        cc = jnp.full((_L,), c, jnp.int32)
        ur = u_rows.at[buf]
        ir = i_rows.at[buf]
        ubr = ub_rows.at[buf]
        ibr = ib_rows.at[buf]

        def group_body(g, carry):
            rows = g * _L + lane
            a0 = jnp.zeros((_L,), jnp.float32)
            a1 = jnp.zeros((_L,), jnp.float32)
            a2 = jnp.zeros((_L,), jnp.float32)
            a3 = jnp.zeros((_L,), jnp.float32)
            for d in range(0, _D, 4):
                c0 = (lane + d) & 63
                c1 = (lane + d + 1) & 63
                c2 = (lane + d + 2) & 63
                c3 = (lane + d + 3) & 63
                a0 = a0 + (plsc.load_gather(ur, [rows, c0]) *
                           plsc.load_gather(ir, [rows, c0 + _D]))
                a1 = a1 + (plsc.load_gather(ur, [rows, c1]) *
                           plsc.load_gather(ir, [rows, c1 + _D]))
                a2 = a2 + (plsc.load_gather(ur, [rows, c2]) *
                           plsc.load_gather(ir, [rows, c2 + _D]))
                a3 = a3 + (plsc.load_gather(ur, [rows, c3]) *
                           plsc.load_gather(ir, [rows, c3 + _D]))
            uq = plsc.load_gather(uidx_v, [cc, rows])
            iq = plsc.load_gather(iidx_v, [cc, rows])
            acc = ((a0 + a1) + (a2 + a3)
                   + plsc.load_gather(ubr, [rows, uq & 15])
                   + plsc.load_gather(ibr, [rows, iq & 15]) + _GLOBAL_MEAN)
            out_v[pl.ds(c * _CHUNK + g * _L, _L)] = acc
            return carry

        lax.fori_loop(0, _CHUNK // _L, group_body, 0)

    compute_bidx(0)
    cps = issue(0, 0)
    for c in range(_NCHUNK):
        nxt = c + 1
        if nxt < _NCHUNK:
            compute_bidx(nxt)
            nxt_cps = issue(nxt, nxt % 2)
        for cp in cps:
            cp.wait()
        compute(c, c % 2)
        if nxt < _NCHUNK:
            cps = nxt_cps

    pltpu.sync_copy(out_v, out_hbm.at[wid])


_mf_call = functools.partial(
    pl.kernel,
    out_type=jax.ShapeDtypeStruct((_NW, _ROWS_PER_TILE), jnp.float32),
    mesh=plsc.VectorSubcoreMesh(core_axis_name="c", subcore_axis_name="s"),
    compiler_params=pltpu.CompilerParams(
        needs_layout_passes=False, use_tc_tiling_on_sc=False),
    scratch_types=[
        pltpu.VMEM((_NCHUNK, _CHUNK), jnp.int32),
        pltpu.VMEM((_NCHUNK, _CHUNK), jnp.int32),
        pltpu.VMEM((_NCHUNK, _CHUNK), jnp.int32),
        pltpu.VMEM((_NCHUNK, _CHUNK), jnp.int32),
        pltpu.VMEM((2, _CHUNK, _W), jnp.float32),
        pltpu.VMEM((2, _CHUNK, _W), jnp.float32),
        pltpu.VMEM((2, _CHUNK, _L), jnp.float32),
        pltpu.VMEM((2, _CHUNK, _L), jnp.float32),
        pltpu.VMEM((_ROWS_PER_TILE,), jnp.float32),
        pltpu.SemaphoreType.DMA,
        pltpu.SemaphoreType.DMA,
    ],
)(_mf_body)


@jax.jit
def kernel(uids, iids, u_factors, i_factors, u_biases, i_biases):
    uids_r = uids.astype(jnp.int32).reshape(_NW, _NCHUNK, _CHUNK)
    iids_r = iids.astype(jnp.int32).reshape(_NW, _NCHUNK, _CHUNK)
    cat = _make_cat(u_factors.shape[0])(u_factors.T, i_factors.T)
    ub_r = u_biases.reshape(-1, _L)
    ib_r = i_biases.reshape(-1, _L)
    out = _mf_call(uids_r, iids_r, cat, ub_r, ib_r)
    return out.reshape(-1)
